# baseline (speedup 1.0000x reference)
import functools

import jax
import jax.numpy as jnp
from jax import lax
from jax.experimental import pallas as pl
from jax.experimental.pallas import tpu as pltpu
from jax.experimental.pallas import tpu_sc as plsc

H = 16
N = 1025
F = 3972
FPAD = 4096
NB = 1024
BR = 16
NBLK = 32
EP = 1152
ER = 1032
EC = 256


def _sc_body(table_hbm, idx_hbm, irow_hbm, icol_hbm,
             out_hbm, erow_hbm, ecol_hbm,
             table_v, idx_v0, idx_v1, out_v0, out_v1,
             erow_v, ecol_v, icol_v,
             insem0, insem1, outsem0, outsem1):
    c = lax.axis_index("c")
    s = lax.axis_index("s")
    wid = s * 2 + c
    head = wid // 2
    half = wid % 2
    base = half * 512
    bufs = ((idx_v0, out_v0, insem0, outsem0),
            (idx_v1, out_v1, insem1, outsem1))

    pltpu.sync_copy(table_hbm.at[head], table_v)

    def in_copy(i, b):
        iv, _, isem, _ = bufs[b]
        pltpu.async_copy(idx_hbm.at[pl.ds(base + 16 * i, BR), pl.ds(0, NB)],
                         iv, isem)

    def in_wait(b):
        iv, _, isem, _ = bufs[b]
        pltpu.make_async_copy(idx_hbm.at[pl.ds(0, BR), pl.ds(0, NB)],
                              iv, isem).wait()

    def out_copy(i, b):
        _, ov, _, osem = bufs[b]
        pltpu.async_copy(ov, out_hbm.at[head, pl.ds(base + 16 * i, BR),
                                        pl.ds(0, NB)], osem)

    def out_wait(b):
        _, ov, _, osem = bufs[b]
        pltpu.make_async_copy(ov, out_hbm.at[0, pl.ds(0, BR), pl.ds(0, NB)],
                              osem).wait()

    in_copy(0, 0)
    in_copy(1, 1)

    lane = lax.iota(jnp.int32, 16)
    zeros16 = jnp.zeros((16,), jnp.int32)

    @pl.when(half == 1)
    def _():
        pltpu.sync_copy(irow_hbm, icol_v)
        for j in range(EP // 16):
            iv = icol_v[pl.ds(j * 16, 16)]
            erow_v[0, pl.ds(j * 16, 16)] = plsc.load_gather(table_v, [iv])
        pltpu.sync_copy(erow_v, erow_hbm.at[head])

    @pl.when(half == 0)
    def _():
        pltpu.sync_copy(icol_hbm, icol_v)
        for k in range(5):
            r0 = min(k * EC, ER - EC)
            for t in range(EC // 16):
                iv = icol_v[pl.ds(r0 + t * 16, 16)]
                vals = plsc.load_gather(table_v, [iv])
                plsc.store_scatter(ecol_v, [lane + t * 16, zeros16], vals)
            pltpu.sync_copy(ecol_v, ecol_hbm.at[head, pl.ds(r0, EC), :])

    def pair(p, _):
        for b in range(2):
            i = 2 * p + b
            iv, ov, _, _ = bufs[b]
            in_wait(b)

            @pl.when(p >= 1)
            def _():
                out_wait(b)

            for r in range(BR):
                @plsc.parallel_loop(0, NB // 16, unroll=8)
                def _(m):
                    ix = iv[r, pl.ds(m * 16, 16)]
                    ov[r, pl.ds(m * 16, 16)] = plsc.load_gather(table_v, [ix])

            out_copy(i, b)

            @pl.when(p <= NBLK // 2 - 2)
            def _():
                in_copy(i + 2, b)
        return 0

    lax.fori_loop(0, NBLK // 2, pair, 0)
    out_wait(0)
    out_wait(1)


def _k1_body(big_ref, erow_ref, out_ref):
    sub = lax.broadcasted_iota(jnp.int32, (1, 8, 128), 1)
    out_ref[...] = jnp.where(sub == 0, erow_ref[...], big_ref[...])


def _k2_body(big_ref, ecol_ref, out_ref):
    lane = lax.broadcasted_iota(jnp.int32, (1, EC, 128), 2)
    out_ref[...] = jnp.where(lane == 0, ecol_ref[...], big_ref[...])


@jax.jit
def _rpb(table_t, idx, irow, icol):
    mesh = plsc.VectorSubcoreMesh(
        core_axis_name="c", subcore_axis_name="s", num_cores=2,
        num_subcores=16)
    bulk, erow, ecol = pl.kernel(
        _sc_body,
        out_type=(
            jax.ShapeDtypeStruct((H, N, N), jnp.float32),
            jax.ShapeDtypeStruct((H, 8, EP), jnp.float32),
            jax.ShapeDtypeStruct((H, ER, 128), jnp.float32),
        ),
        mesh=mesh,
        compiler_params=pltpu.CompilerParams(
            needs_layout_passes=False, use_tc_tiling_on_sc=True),
        scratch_types=[
            pltpu.VMEM((FPAD,), jnp.float32),
            pltpu.VMEM((BR, NB), jnp.int32),
            pltpu.VMEM((BR, NB), jnp.int32),
            pltpu.VMEM((BR, NB), jnp.float32),
            pltpu.VMEM((BR, NB), jnp.float32),
            pltpu.VMEM((8, EP), jnp.float32),
            pltpu.VMEM((EC, 128), jnp.float32),
            pltpu.VMEM((EP,), jnp.int32),
            pltpu.SemaphoreType.DMA,
            pltpu.SemaphoreType.DMA,
            pltpu.SemaphoreType.DMA,
            pltpu.SemaphoreType.DMA,
        ],
    )(table_t, idx, irow, icol)

    with_row = pl.pallas_call(
        _k1_body,
        out_shape=jax.ShapeDtypeStruct((H, N, N), jnp.float32),
        grid=(H, 9),
        in_specs=[
            pl.BlockSpec((1, 8, 128), lambda h, t: (h, 128, t)),
            pl.BlockSpec((1, 8, 128), lambda h, t: (h, 0, t)),
        ],
        out_specs=pl.BlockSpec((1, 8, 128), lambda h, t: (h, 128, t)),
        input_output_aliases={0: 0},
    )(bulk, erow)

    out = pl.pallas_call(
        _k2_body,
        out_shape=jax.ShapeDtypeStruct((H, N, N), jnp.float32),
        grid=(H, 5),
        in_specs=[
            pl.BlockSpec((1, EC, 128), lambda h, s: (h, s, 8)),
            pl.BlockSpec((1, EC, 128), lambda h, s: (h, s, 0)),
        ],
        out_specs=pl.BlockSpec((1, EC, 128), lambda h, s: (h, s, 8)),
        input_output_aliases={0: 0},
    )(with_row, ecol)
    return out


def kernel(relative_position_bias_table, relative_position_index):
    table_t = jnp.pad(relative_position_bias_table.T, ((0, 0), (0, FPAD - F)))
    irow = jnp.pad(relative_position_index[N - 1, :], (0, EP - N))
    icol = jnp.pad(relative_position_index[:, N - 1], (0, EP - N))
    return _rpb(table_t, relative_position_index, irow, icol)


# speedup vs baseline: 1.6078x; 1.6078x over previous
import functools

import jax
import jax.numpy as jnp
from jax import lax
from jax.experimental import pallas as pl
from jax.experimental.pallas import tpu as pltpu
from jax.experimental.pallas import tpu_sc as plsc

H = 16
N = 1025
F = 3972
FPAD = 4096
NB = 1024
BR = 16
NBLK = 32
EP = 1152
ER = 1032
EC = 256


def _sc_body(table_hbm, idx_hbm, irow_hbm, icol_hbm, out_hbm,
             table_v, idx_v0, idx_v1, out_v0, out_v1,
             erow_v, ecol_v, icol_v,
             insem0, insem1, outsem0, outsem1):
    c = lax.axis_index("c")
    s = lax.axis_index("s")
    wid = s * 2 + c
    head = wid // 2
    half = wid % 2
    base = half * 512
    bufs = ((idx_v0, out_v0, insem0, outsem0),
            (idx_v1, out_v1, insem1, outsem1))

    pltpu.sync_copy(table_hbm.at[head], table_v)

    def in_copy(i, b):
        iv, _, isem, _ = bufs[b]
        pltpu.async_copy(idx_hbm.at[pl.ds(base + 16 * i, BR), pl.ds(0, NB)],
                         iv, isem)

    def in_wait(b):
        iv, _, isem, _ = bufs[b]
        pltpu.make_async_copy(idx_hbm.at[pl.ds(0, BR), pl.ds(0, NB)],
                              iv, isem).wait()

    def out_copy(i, b):
        _, ov, _, osem = bufs[b]
        pltpu.async_copy(ov, out_hbm.at[head, pl.ds(base + 16 * i, BR),
                                        pl.ds(0, NB)], osem)

    def out_wait(b):
        _, ov, _, osem = bufs[b]
        pltpu.make_async_copy(ov, out_hbm.at[0, pl.ds(0, BR), pl.ds(0, NB)],
                              osem).wait()

    in_copy(0, 0)
    in_copy(1, 1)

    lane = lax.iota(jnp.int32, 16)
    zeros16 = jnp.zeros((16,), jnp.int32)

    @pl.when(half == 1)
    def _():
        pltpu.sync_copy(irow_hbm, icol_v)
        for j in range(NB // 16):
            iv = icol_v[pl.ds(j * 16, 16)]
            erow_v[0, pl.ds(j * 16, 16)] = plsc.load_gather(table_v, [iv])
        pltpu.sync_copy(erow_v, out_hbm.at[head, pl.ds(NB, 8), pl.ds(0, NB)])

    @pl.when(half == 0)
    def _():
        pltpu.sync_copy(icol_hbm, icol_v)
        for k in range(5):
            r0 = min(k * EC, 1032 - EC)
            for t in range(EC // 16):
                iv = icol_v[pl.ds(r0 + t * 16, 16)]
                vals = plsc.load_gather(table_v, [iv])
                plsc.store_scatter(ecol_v, [lane + t * 16, zeros16], vals)
            pltpu.sync_copy(ecol_v,
                            out_hbm.at[head, pl.ds(r0, EC), pl.ds(NB, 128)])


    def pair(p, _):
        for b in range(2):
            i = 2 * p + b
            iv, ov, _, _ = bufs[b]
            in_wait(b)

            @pl.when(p >= 1)
            def _():
                out_wait(b)

            for r in range(BR):
                @plsc.parallel_loop(0, NB // 16, unroll=8)
                def _(m):
                    ix = iv[r, pl.ds(m * 16, 16)]
                    ov[r, pl.ds(m * 16, 16)] = plsc.load_gather(table_v, [ix])

            out_copy(i, b)

            @pl.when(p <= NBLK // 2 - 2)
            def _():
                in_copy(i + 2, b)
        return 0

    lax.fori_loop(0, NBLK // 2, pair, 0)
    out_wait(0)
    out_wait(1)


@jax.jit
def _rpb(table_t, idx, irow, icol):
    mesh = plsc.VectorSubcoreMesh(
        core_axis_name="c", subcore_axis_name="s", num_cores=2,
        num_subcores=16)
    return pl.kernel(
        _sc_body,
        out_type=jax.ShapeDtypeStruct((H, ER, EP), jnp.float32),
        mesh=mesh,
        compiler_params=pltpu.CompilerParams(
            needs_layout_passes=False, use_tc_tiling_on_sc=True),
        scratch_types=[
            pltpu.VMEM((FPAD,), jnp.float32),
            pltpu.VMEM((BR, NB), jnp.int32),
            pltpu.VMEM((BR, NB), jnp.int32),
            pltpu.VMEM((BR, NB), jnp.float32),
            pltpu.VMEM((BR, NB), jnp.float32),
            pltpu.VMEM((8, NB), jnp.float32),
            pltpu.VMEM((EC, 128), jnp.float32),
            pltpu.VMEM((EP,), jnp.int32),
            pltpu.SemaphoreType.DMA,
            pltpu.SemaphoreType.DMA,
            pltpu.SemaphoreType.DMA,
            pltpu.SemaphoreType.DMA,
        ],
    )(table_t, idx, irow, icol)


def kernel(relative_position_bias_table, relative_position_index):
    table_t = jnp.pad(relative_position_bias_table.T, ((0, 0), (0, FPAD - F)))
    irow = jnp.pad(relative_position_index[N - 1, :], (0, EP - N))
    icol = jnp.pad(relative_position_index[:, N - 1], (0, EP - N))
    out_p = _rpb(table_t, relative_position_index, irow, icol)
    return out_p[:, :N, :N]
